# baseline (speedup 1.0000x reference)
import functools

import jax
import jax.numpy as jnp
from jax import lax
from jax.experimental import pallas as pl
from jax.experimental.pallas import tpu as pltpu
from jax.experimental.pallas import tpu_sc as plsc

_B = 1024
_L = 200
_H = 128
_VOCAB = 100000
_EPS = 1e-12

_NW = 32
_SEQ_PER_W = _B // _NW
_CH = 40
_NCH = _B * _L // _CH
_CH_PER_W = _NCH // _NW


def _allsum(v):
    dnums = lax.GatherDimensionNumbers(
        offset_dims=(), collapsed_slice_dims=(0,), start_index_map=(0,))
    lane = lax.iota(jnp.int32, 16)
    for sh in (1, 2, 4, 8):
        idx = ((lane + sh) & 15).reshape(16, 1)
        rot = lax.gather(v, idx, dnums, slice_sizes=(1,),
                         mode=lax.GatherScatterMode.PROMISE_IN_BOUNDS)
        v = v + rot
    return v


def _ln_row(buf, pos_v, w_v, b_v, r, pos_r):
    xs = []
    for j in range(_H // 16):
        x = buf[r, pl.ds(j * 16, 16)]
        p = pos_v[pos_r, pl.ds(j * 16, 16)]
        xs.append(x + p)
    acc = xs[0]
    acc2 = xs[0] * xs[0]
    for j in range(1, _H // 16):
        acc = acc + xs[j]
        acc2 = acc2 + xs[j] * xs[j]
    tot = _allsum(acc)
    tot2 = _allsum(acc2)
    mean = tot * (1.0 / _H)
    ex2 = tot2 * (1.0 / _H)
    a = ex2 - mean * mean + _EPS
    bits = lax.bitcast_convert_type(a, jnp.int32)
    seed = jnp.int32(0x5F3759DF) - lax.shift_right_logical(bits, 1)
    y = lax.bitcast_convert_type(seed, jnp.float32)
    half = a * 0.5
    for _ in range(3):
        y = y * (1.5 - half * y * y)
    for j in range(_H // 16):
        w = w_v[pl.ds(j * 16, 16)]
        b = b_v[pl.ds(j * 16, 16)]
        t = (xs[j] - mean) * y
        buf[r, pl.ds(j * 16, 16)] = t * w + b


_NBUF = 8


def _sc_body(ids_hbm, table_hbm, postt_hbm, w_hbm, b_hbm, out_hbm,
             idx_v, pos_v, w_v, b_v, *bufs_and_sems):
    nc = 2
    wid = lax.axis_index("s") * nc + lax.axis_index("c")
    bufs = bufs_and_sems[:_NBUF]
    gsems = bufs_and_sems[_NBUF:2 * _NBUF]
    osems = bufs_and_sems[2 * _NBUF:]
    base_chunk = wid * _CH_PER_W

    pltpu.sync_copy(ids_hbm.at[pl.ds(base_chunk, _CH_PER_W)], idx_v)
    pltpu.sync_copy(postt_hbm, pos_v)
    pltpu.sync_copy(w_hbm, w_v)
    pltpu.sync_copy(b_hbm, b_v)

    def gather(k, c):
        pltpu.async_copy(table_hbm.at[idx_v.at[c]], bufs[k], gsems[k])

    def wait_gather(k, c):
        pltpu.make_async_copy(table_hbm.at[idx_v.at[c]], bufs[k],
                              gsems[k]).wait()

    def scatter(k, c):
        row0 = (base_chunk + c) * _CH
        pltpu.async_copy(bufs[k], out_hbm.at[pl.ds(row0, _CH)], osems[k])

    def wait_scatter(k, c):
        row0 = (base_chunk + c) * _CH
        pltpu.make_async_copy(bufs[k], out_hbm.at[pl.ds(row0, _CH)],
                              osems[k]).wait()

    def compute(k, c):
        poff = lax.rem(c, 5) * _CH

        def row_step(r, carry):
            _ln_row(bufs[k], pos_v, w_v, b_v, r, poff + r)
            return carry

        lax.fori_loop(0, _CH, row_step, 0)

    for k in range(_NBUF):
        gather(k, k)

    nsteps = _CH_PER_W // _NBUF

    def step(t, carry):
        for k in range(_NBUF):
            c = t * _NBUF + k
            wait_gather(k, c)
            compute(k, c)
            scatter(k, c)
        for k in range(_NBUF):
            c = t * _NBUF + k

            @pl.when(t < nsteps - 1)
            def _():
                wait_scatter(k, c)
                gather(k, c + _NBUF)

        return carry

    lax.fori_loop(0, nsteps, step, 0)
    for k in range(_NBUF):
        wait_scatter(k, _CH_PER_W - _NBUF + k)


@jax.jit
def _camembert_sc(ids2d, table, postt, w, b):
    mesh = plsc.VectorSubcoreMesh(core_axis_name="c", subcore_axis_name="s")
    run = pl.kernel(
        _sc_body,
        out_type=jax.ShapeDtypeStruct((_B * _L, _H), jnp.float32),
        mesh=mesh,
        scratch_types=(
            [pltpu.VMEM((_CH_PER_W, _CH), jnp.int32),
             pltpu.VMEM((_L, _H), jnp.float32),
             pltpu.VMEM((_H,), jnp.float32),
             pltpu.VMEM((_H,), jnp.float32)]
            + [pltpu.VMEM((_CH, _H), jnp.float32) for _ in range(_NBUF)]
            + [pltpu.SemaphoreType.DMA for _ in range(2 * _NBUF)]
        ),
    )
    return run(ids2d, table, postt, w, b)


def kernel(input_ids, word_embeddings, position_embeddings,
           token_type_embeddings, ln_weight, ln_bias):
    ids2d = input_ids.astype(jnp.int32).reshape(_NCH, _CH)
    postt = position_embeddings[:_L] + token_type_embeddings[0]
    out = _camembert_sc(ids2d, word_embeddings, postt, ln_weight, ln_bias)
    return out.reshape(_B, _L, _H)


# speedup vs baseline: 1.7578x; 1.7578x over previous
import functools

import jax
import jax.numpy as jnp
from jax import lax
from jax.experimental import pallas as pl
from jax.experimental.pallas import tpu as pltpu
from jax.experimental.pallas import tpu_sc as plsc

_B = 1024
_L = 200
_H = 128
_VOCAB = 100000
_EPS = 1e-12

_NW = 32
_SEQ_PER_W = _B // _NW
_CH = 40
_NCH = _B * _L // _CH
_CH_PER_W = _NCH // _NW


def _allsum(v):
    dnums = lax.GatherDimensionNumbers(
        offset_dims=(), collapsed_slice_dims=(0,), start_index_map=(0,))
    lane = lax.iota(jnp.int32, 16)
    for sh in (1, 2, 4, 8):
        idx = ((lane + sh) & 15).reshape(16, 1)
        rot = lax.gather(v, idx, dnums, slice_sizes=(1,),
                         mode=lax.GatherScatterMode.PROMISE_IN_BOUNDS)
        v = v + rot
    return v


def _ln_row(buf, pos_v, w_v, b_v, r, pos_r):
    xs = []
    for j in range(_H // 16):
        x = buf[r, pl.ds(j * 16, 16)]
        p = pos_v[pos_r, pl.ds(j * 16, 16)]
        xs.append(x + p)
    acc = xs[0]
    acc2 = xs[0] * xs[0]
    for j in range(1, _H // 16):
        acc = acc + xs[j]
        acc2 = acc2 + xs[j] * xs[j]
    tot = _allsum(acc)
    tot2 = _allsum(acc2)
    mean = tot * (1.0 / _H)
    ex2 = tot2 * (1.0 / _H)
    a = ex2 - mean * mean + _EPS
    bits = lax.bitcast_convert_type(a, jnp.int32)
    seed = jnp.int32(0x5F3759DF) - lax.shift_right_logical(bits, 1)
    y = lax.bitcast_convert_type(seed, jnp.float32)
    half = a * 0.5
    for _ in range(3):
        y = y * (1.5 - half * y * y)
    for j in range(_H // 16):
        w = w_v[pl.ds(j * 16, 16)]
        b = b_v[pl.ds(j * 16, 16)]
        t = (xs[j] - mean) * y
        buf[r, pl.ds(j * 16, 16)] = t * w + b


_NBUF = 4


def _sc_body(ids_hbm, table_hbm, postt_hbm, w_hbm, b_hbm, out_hbm,
             idx_v, pos_v, w_v, b_v, *bufs_and_sems):
    nc = 2
    wid = lax.axis_index("s") * nc + lax.axis_index("c")
    bufs = bufs_and_sems[:_NBUF]
    gsems = bufs_and_sems[_NBUF:2 * _NBUF]
    osems = bufs_and_sems[2 * _NBUF:]
    base_chunk = wid * _CH_PER_W

    pltpu.sync_copy(ids_hbm.at[pl.ds(base_chunk, _CH_PER_W)], idx_v)
    pltpu.sync_copy(postt_hbm, pos_v)
    pltpu.sync_copy(w_hbm, w_v)
    pltpu.sync_copy(b_hbm, b_v)

    def gather(k, c):
        pltpu.async_copy(table_hbm.at[idx_v.at[c]], bufs[k], gsems[k])

    def wait_gather(k, c):
        pltpu.make_async_copy(table_hbm.at[idx_v.at[c]], bufs[k],
                              gsems[k]).wait()

    def scatter(k, c):
        row0 = (base_chunk + c) * _CH
        pltpu.async_copy(bufs[k], out_hbm.at[pl.ds(row0, _CH)], osems[k])

    def wait_scatter(k, c):
        row0 = (base_chunk + c) * _CH
        pltpu.make_async_copy(bufs[k], out_hbm.at[pl.ds(row0, _CH)],
                              osems[k]).wait()

    def compute(k, c):
        poff = lax.rem(c, 5) * _CH

        @plsc.parallel_loop(0, _CH, unroll=2)
        def row_step(r):
            _ln_row(bufs[k], pos_v, w_v, b_v, r, poff + r)

    for k in range(_NBUF):
        gather(k, k)

    nsteps = _CH_PER_W // _NBUF

    def step(t, carry):
        for k in range(_NBUF):
            c = t * _NBUF + k
            wait_gather(k, c)
            compute(k, c)
            scatter(k, c)
        for k in range(_NBUF):
            c = t * _NBUF + k

            @pl.when(t < nsteps - 1)
            def _():
                wait_scatter(k, c)
                gather(k, c + _NBUF)

        return carry

    lax.fori_loop(0, nsteps, step, 0)
    for k in range(_NBUF):
        wait_scatter(k, _CH_PER_W - _NBUF + k)


@jax.jit
def _camembert_sc(ids2d, table, postt, w, b):
    mesh = plsc.VectorSubcoreMesh(core_axis_name="c", subcore_axis_name="s")
    run = pl.kernel(
        _sc_body,
        out_type=jax.ShapeDtypeStruct((_B * _L, _H), jnp.float32),
        mesh=mesh,
        scratch_types=(
            [pltpu.VMEM((_CH_PER_W, _CH), jnp.int32),
             pltpu.VMEM((_L, _H), jnp.float32),
             pltpu.VMEM((_H,), jnp.float32),
             pltpu.VMEM((_H,), jnp.float32)]
            + [pltpu.VMEM((_CH, _H), jnp.float32) for _ in range(_NBUF)]
            + [pltpu.SemaphoreType.DMA for _ in range(2 * _NBUF)]
        ),
    )
    return run(ids2d, table, postt, w, b)


def kernel(input_ids, word_embeddings, position_embeddings,
           token_type_embeddings, ln_weight, ln_bias):
    ids2d = input_ids.astype(jnp.int32).reshape(_NCH, _CH)
    postt = position_embeddings[:_L] + token_type_embeddings[0]
    out = _camembert_sc(ids2d, word_embeddings, postt, ln_weight, ln_bias)
    return out.reshape(_B, _L, _H)


# speedup vs baseline: 2.7770x; 1.5798x over previous
import functools

import jax
import jax.numpy as jnp
from jax import lax
from jax.experimental import pallas as pl
from jax.experimental.pallas import tpu as pltpu
from jax.experimental.pallas import tpu_sc as plsc

_B = 1024
_L = 200
_H = 128
_VOCAB = 100000
_EPS = 1e-12

_NW = 32
_SEQ_PER_W = _B // _NW
_CH = 40
_NCH = _B * _L // _CH
_CH_PER_W = _NCH // _NW


def _allsum(v):
    dnums = lax.GatherDimensionNumbers(
        offset_dims=(), collapsed_slice_dims=(0,), start_index_map=(0,))
    lane = lax.iota(jnp.int32, 16)
    for sh in (1, 2, 4, 8):
        idx = ((lane + sh) & 15).reshape(16, 1)
        rot = lax.gather(v, idx, dnums, slice_sizes=(1,),
                         mode=lax.GatherScatterMode.PROMISE_IN_BOUNDS)
        v = v + rot
    return v


def _ln_row(buf, pos_v, ws, bs, r, pos_r):
    xs = []
    for j in range(_H // 16):
        x = buf[r, pl.ds(j * 16, 16)]
        p = pos_v[pos_r, pl.ds(j * 16, 16)]
        xs.append(x + p)
    acc = xs[0]
    acc2 = xs[0] * xs[0]
    for j in range(1, _H // 16):
        acc = acc + xs[j]
        acc2 = acc2 + xs[j] * xs[j]
    tot = _allsum(acc)
    tot2 = _allsum(acc2)
    mean = tot * (1.0 / _H)
    ex2 = tot2 * (1.0 / _H)
    a = ex2 - mean * mean + _EPS
    bits = lax.bitcast_convert_type(a, jnp.int32)
    seed = jnp.int32(0x5F3759DF) - lax.shift_right_logical(bits, 1)
    y = lax.bitcast_convert_type(seed, jnp.float32)
    half = a * 0.5
    for _ in range(2):
        y = y * (1.5 - half * y * y)
    for j in range(_H // 16):
        t = (xs[j] - mean) * y
        buf[r, pl.ds(j * 16, 16)] = t * ws[j] + bs[j]


_NBUF = 4


def _sc_body(ids_hbm, table_hbm, postt_hbm, w_hbm, b_hbm, out_hbm,
             idx_v, pos_v, w_v, b_v, *bufs_and_sems):
    nc = 2
    wid = lax.axis_index("s") * nc + lax.axis_index("c")
    bufs = bufs_and_sems[:_NBUF]
    gsems = bufs_and_sems[_NBUF:2 * _NBUF]
    osems = bufs_and_sems[2 * _NBUF:]
    base_chunk = wid * _CH_PER_W

    pltpu.sync_copy(ids_hbm.at[pl.ds(base_chunk, _CH_PER_W)], idx_v)
    pltpu.sync_copy(postt_hbm, pos_v)
    pltpu.sync_copy(w_hbm, w_v)
    pltpu.sync_copy(b_hbm, b_v)

    def gather(k, c):
        pltpu.async_copy(table_hbm.at[idx_v.at[c]], bufs[k], gsems[k])

    def wait_gather(k, c):
        pltpu.make_async_copy(table_hbm.at[idx_v.at[c]], bufs[k],
                              gsems[k]).wait()

    def scatter(k, c):
        row0 = (base_chunk + c) * _CH
        pltpu.async_copy(bufs[k], out_hbm.at[pl.ds(row0, _CH)], osems[k])

    def wait_scatter(k, c):
        row0 = (base_chunk + c) * _CH
        pltpu.make_async_copy(bufs[k], out_hbm.at[pl.ds(row0, _CH)],
                              osems[k]).wait()

    ws = [w_v[pl.ds(j * 16, 16)] for j in range(_H // 16)]
    bs = [b_v[pl.ds(j * 16, 16)] for j in range(_H // 16)]

    def compute(k, c):
        poff = lax.rem(c, 5) * _CH

        @plsc.parallel_loop(0, _CH, unroll=2)
        def row_step(r):
            _ln_row(bufs[k], pos_v, ws, bs, r, poff + r)

    for k in range(_NBUF):
        gather(k, k)

    nsteps = _CH_PER_W // _NBUF

    def step(t, carry):
        for k in range(_NBUF):
            c = t * _NBUF + k
            wait_gather(k, c)
            compute(k, c)
            scatter(k, c)
        for k in range(_NBUF):
            c = t * _NBUF + k

            @pl.when(t < nsteps - 1)
            def _():
                wait_scatter(k, c)
                gather(k, c + _NBUF)

        return carry

    lax.fori_loop(0, nsteps, step, 0)
    for k in range(_NBUF):
        wait_scatter(k, _CH_PER_W - _NBUF + k)


@jax.jit
def _camembert_sc(ids2d, table, postt, w, b):
    mesh = plsc.VectorSubcoreMesh(core_axis_name="c", subcore_axis_name="s")
    run = pl.kernel(
        _sc_body,
        out_type=jax.ShapeDtypeStruct((_B * _L, _H), jnp.float32),
        mesh=mesh,
        scratch_types=(
            [pltpu.VMEM((_CH_PER_W, _CH), jnp.int32),
             pltpu.VMEM((_L, _H), jnp.float32),
             pltpu.VMEM((_H,), jnp.float32),
             pltpu.VMEM((_H,), jnp.float32)]
            + [pltpu.VMEM((_CH, _H), jnp.float32) for _ in range(_NBUF)]
            + [pltpu.SemaphoreType.DMA for _ in range(2 * _NBUF)]
        ),
    )
    return run(ids2d, table, postt, w, b)


def kernel(input_ids, word_embeddings, position_embeddings,
           token_type_embeddings, ln_weight, ln_bias):
    ids2d = input_ids.astype(jnp.int32).reshape(_NCH, _CH)
    postt = position_embeddings[:_L] + token_type_embeddings[0]
    out = _camembert_sc(ids2d, word_embeddings, postt, ln_weight, ln_bias)
    return out.reshape(_B, _L, _H)
